# baseline (speedup 1.0000x reference)
import functools

import jax
import jax.numpy as jnp
from jax import lax
from jax.experimental import pallas as pl
from jax.experimental.pallas import tpu as pltpu
from jax.experimental.pallas import tpu_sc as plsc

N = 65536
C = 12
CP = 16
PACK = 4 * CP
NC, NS, L = 2, 16, 16
NW = NC * NS
BPW = N // NW
CH = 128
NCHUNK = BPW // CH
G = CH // L

HXY = 128
HUV = 256
BN = 512


def _pack_table(plane):
    H, W, _ = plane.shape
    p = jnp.pad(plane, ((0, 0), (0, 0), (0, CP - C)))
    t = jnp.concatenate([p[:-1, :-1], p[:-1, 1:], p[1:, :-1], p[1:, 1:]], axis=-1)
    return t.reshape((H - 1) * (W - 1), PACK)


def _coords(xy_v, b, s):
    x = jnp.clip(xy_v[b, 0, pl.ds(s, L)], 0.0, 1.0) * float(HXY - 1)
    y = jnp.clip(xy_v[b, 1, pl.ds(s, L)], 0.0, 1.0) * float(HXY - 1)
    u = jnp.clip(xy_v[b, 2, pl.ds(s, L)], 0.0, 1.0) * float(HUV - 1)
    v = jnp.clip(xy_v[b, 3, pl.ds(s, L)], 0.0, 1.0) * float(HUV - 1)
    xi = jnp.minimum(x.astype(jnp.int32), HXY - 2)
    yi = jnp.minimum(y.astype(jnp.int32), HXY - 2)
    ui = jnp.minimum(u.astype(jnp.int32), HUV - 2)
    vi = jnp.minimum(v.astype(jnp.int32), HUV - 2)
    return x, y, u, v, xi, yi, ui, vi


def _sc_body(xyuv_hbm, txy_hbm, tuv_hbm, out_hbm,
             xy_v, ixy_v, iuv_v, rxy_v, ruv_v, k0_v,
             sxy0, sxy1, suv0, suv1):
    wid = lax.axis_index("s") * NC + lax.axis_index("c")
    base = wid * BPW
    zeros = jnp.zeros((L,), jnp.float32)
    sems_xy = (sxy0, sxy1)
    sems_uv = (suv0, suv1)

    def zb(g, carry):
        for c in range(C, CP):
            k0_v[c, pl.ds(g * L, L)] = zeros
        return carry
    lax.fori_loop(0, G, zb, 0)

    def stage(ci, b):
        cb = base + ci * CH
        with jax.named_scope("sc_xyld"):
            pltpu.sync_copy(xyuv_hbm.at[:, pl.ds(cb, CH)], xy_v.at[b])

        def idxb(g, c2):
            s = g * L
            _, _, _, _, xi, yi, ui, vi = _coords(xy_v, b, s)
            ixy_v[b, pl.ds(s, L)] = xi * (HXY - 1) + yi
            iuv_v[b, pl.ds(s, L)] = ui * (HUV - 1) + vi
            return c2
        with jax.named_scope("sc_idx"):
            lax.fori_loop(0, G, idxb, 0)

        with jax.named_scope("sc_fire"):
            pltpu.async_copy(txy_hbm.at[ixy_v.at[b]], rxy_v.at[b], sems_xy[b])
            pltpu.async_copy(tuv_hbm.at[iuv_v.at[b]], ruv_v.at[b], sems_uv[b])

    def consume(ci, b):
        cb = base + ci * CH
        with jax.named_scope("sc_wait"):
            pltpu.make_async_copy(txy_hbm.at[ixy_v.at[b]], rxy_v.at[b],
                                  sems_xy[b]).wait()
            pltpu.make_async_copy(tuv_hbm.at[iuv_v.at[b]], ruv_v.at[b],
                                  sems_uv[b]).wait()

        def ib(g, c2):
            s = g * L
            x, y, u, v, xi, yi, ui, vi = _coords(xy_v, b, s)
            wx = x - xi.astype(jnp.float32)
            wy = y - yi.astype(jnp.float32)
            wu = u - ui.astype(jnp.float32)
            wv = v - vi.astype(jnp.float32)
            w00 = (1.0 - wx) * (1.0 - wy)
            w01 = (1.0 - wx) * wy
            w10 = wx * (1.0 - wy)
            w11 = wx * wy
            a00 = (1.0 - wu) * (1.0 - wv)
            a01 = (1.0 - wu) * wv
            a10 = wu * (1.0 - wv)
            a11 = wu * wv
            rows = s + lax.iota(jnp.int32, L)
            rxy_b = rxy_v.at[b]
            ruv_b = ruv_v.at[b]
            for c in range(C):
                i0 = jnp.full((L,), c, jnp.int32)
                i1 = jnp.full((L,), c + CP, jnp.int32)
                i2 = jnp.full((L,), c + 2 * CP, jnp.int32)
                i3 = jnp.full((L,), c + 3 * CP, jnp.int32)
                fxy = (plsc.load_gather(rxy_b, [rows, i0]) * w00
                       + plsc.load_gather(rxy_b, [rows, i1]) * w01
                       + plsc.load_gather(rxy_b, [rows, i2]) * w10
                       + plsc.load_gather(rxy_b, [rows, i3]) * w11)
                fuv = (plsc.load_gather(ruv_b, [rows, i0]) * a00
                       + plsc.load_gather(ruv_b, [rows, i1]) * a01
                       + plsc.load_gather(ruv_b, [rows, i2]) * a10
                       + plsc.load_gather(ruv_b, [rows, i3]) * a11)
                k0_v[c, pl.ds(s, L)] = fxy * fuv
            return c2
        with jax.named_scope("sc_interp"):
            lax.fori_loop(0, G, ib, 0, unroll=4)

        with jax.named_scope("sc_k0st"):
            pltpu.sync_copy(k0_v, out_hbm.at[:, pl.ds(cb, CH)])

    stage(0, 0)

    def chunk2(cj, carry):
        for b in range(2):
            ci = cj * 2 + b

            @pl.when(ci + 1 < NCHUNK)
            def _():
                stage(ci + 1, 1 - b)
            consume(ci, b)
        return carry
    lax.fori_loop(0, NCHUNK // 2, chunk2, 0)


@functools.cache
def _sc_interp():
    return functools.partial(
        pl.kernel,
        out_type=jax.ShapeDtypeStruct((CP, N), jnp.float32),
        mesh=plsc.VectorSubcoreMesh(core_axis_name="c", subcore_axis_name="s",
                                    num_cores=NC, num_subcores=NS),
        scratch_types=[
            pltpu.VMEM((2, 4, CH), jnp.float32),
            pltpu.VMEM((2, CH), jnp.int32),
            pltpu.VMEM((2, CH), jnp.int32),
            pltpu.VMEM((2, CH, PACK), jnp.float32),
            pltpu.VMEM((2, CH, PACK), jnp.float32),
            pltpu.VMEM((CP, CH), jnp.float32),
            pltpu.SemaphoreType.DMA,
            pltpu.SemaphoreType.DMA,
            pltpu.SemaphoreType.DMA,
            pltpu.SemaphoreType.DMA,
        ],
        compiler_params=pltpu.CompilerParams(needs_layout_passes=False,
                                             use_tc_tiling_on_sc=False,
                                             disable_bounds_checks=True),
    )(_sc_body)


def _mlp_body(x_ref, w1_ref, b1_ref, w2_ref, b2_ref, w3_ref, b3_ref, o_ref):
    x = x_ref[...]
    h = lax.dot_general(x, w1_ref[...], (((0,), (0,)), ((), ())),
                        preferred_element_type=jnp.float32)
    h = jnp.maximum(h + b1_ref[...], 0.0)
    h = jnp.maximum(jnp.dot(h, w2_ref[...], preferred_element_type=jnp.float32)
                    + b2_ref[...], 0.0)
    o = jnp.dot(h, w3_ref[...], preferred_element_type=jnp.float32) + b3_ref[...]
    o_ref[...] = jax.nn.sigmoid(o)


_mlp = pl.pallas_call(
    _mlp_body,
    grid=(N // BN,),
    in_specs=[
        pl.BlockSpec((CP, BN), lambda i: (0, i)),
        pl.BlockSpec((CP, 128), lambda i: (0, 0)),
        pl.BlockSpec((1, 128), lambda i: (0, 0)),
        pl.BlockSpec((128, 128), lambda i: (0, 0)),
        pl.BlockSpec((1, 128), lambda i: (0, 0)),
        pl.BlockSpec((128, 8), lambda i: (0, 0)),
        pl.BlockSpec((1, 8), lambda i: (0, 0)),
    ],
    out_specs=pl.BlockSpec((BN, 8), lambda i: (i, 0)),
    out_shape=jax.ShapeDtypeStruct((N, 8), jnp.float32),
)


def kernel(xyuv, plane_xy, plane_uv, W1, b1, W2, b2, W3, b3):
    xyuv_t = xyuv.T
    txy = _pack_table(plane_xy)
    tuv = _pack_table(plane_uv)
    k0t = _sc_interp()(xyuv_t, txy, tuv)
    w1p = jnp.pad(W1, ((0, CP - C), (0, 0)))
    w3p = jnp.pad(W3, ((0, 0), (0, 8 - 3)))
    b3p = jnp.pad(b3, (0, 8 - 3))
    out = _mlp(k0t, w1p, b1.reshape(1, -1), W2, b2.reshape(1, -1),
               w3p, b3p.reshape(1, -1))
    return out[:, :3]


# speedup vs baseline: 1.0210x; 1.0210x over previous
import functools

import jax
import jax.numpy as jnp
from jax import lax
from jax.experimental import pallas as pl
from jax.experimental.pallas import tpu as pltpu
from jax.experimental.pallas import tpu_sc as plsc

N = 65536
C = 12
CP = 16
PACK = 4 * CP
NC, NS, L = 2, 16, 16
NW = NC * NS
BPW = N // NW
CH = 128
NCHUNK = BPW // CH
G = CH // L

HXY = 128
HUV = 256
BN = 512


def _pack_table(plane):
    H, W, _ = plane.shape
    p = jnp.pad(plane, ((0, 0), (0, 0), (0, CP - C)))
    t = jnp.concatenate([p[:-1, :-1], p[:-1, 1:], p[1:, :-1], p[1:, 1:]], axis=-1)
    return t.reshape((H - 1) * (W - 1), PACK)


def _coords(xy_v, b, s):
    x = jnp.clip(xy_v[b, 0, pl.ds(s, L)], 0.0, 1.0) * float(HXY - 1)
    y = jnp.clip(xy_v[b, 1, pl.ds(s, L)], 0.0, 1.0) * float(HXY - 1)
    u = jnp.clip(xy_v[b, 2, pl.ds(s, L)], 0.0, 1.0) * float(HUV - 1)
    v = jnp.clip(xy_v[b, 3, pl.ds(s, L)], 0.0, 1.0) * float(HUV - 1)
    xi = jnp.minimum(x.astype(jnp.int32), HXY - 2)
    yi = jnp.minimum(y.astype(jnp.int32), HXY - 2)
    ui = jnp.minimum(u.astype(jnp.int32), HUV - 2)
    vi = jnp.minimum(v.astype(jnp.int32), HUV - 2)
    return x, y, u, v, xi, yi, ui, vi


def _sc_body(xyuv_hbm, txy_hbm, tuv_hbm, out_hbm,
             xy_v, ixy_v, iuv_v, rxy_v, ruv_v, k0_v,
             sxy0, sxy1, suv0, suv1):
    wid = lax.axis_index("s") * NC + lax.axis_index("c")
    base = wid * BPW
    zeros = jnp.zeros((L,), jnp.float32)
    sems_xy = (sxy0, sxy1)
    sems_uv = (suv0, suv1)

    def zb(g, carry):
        for c in range(C, CP):
            k0_v[c, pl.ds(g * L, L)] = zeros
        return carry
    lax.fori_loop(0, G, zb, 0)

    def stage(ci, b):
        cb = base + ci * CH
        with jax.named_scope("sc_xyld"):
            pltpu.sync_copy(xyuv_hbm.at[:, pl.ds(cb, CH)], xy_v.at[b])

        def idxb(g):
            s = g * L
            _, _, _, _, xi, yi, ui, vi = _coords(xy_v, b, s)
            ixy_v[b, pl.ds(s, L)] = xi * (HXY - 1) + yi
            iuv_v[b, pl.ds(s, L)] = ui * (HUV - 1) + vi
        with jax.named_scope("sc_idx"):
            plsc.parallel_loop(0, G, unroll=2)(idxb)

        with jax.named_scope("sc_fire"):
            pltpu.async_copy(txy_hbm.at[ixy_v.at[b]], rxy_v.at[b], sems_xy[b])
            pltpu.async_copy(tuv_hbm.at[iuv_v.at[b]], ruv_v.at[b], sems_uv[b])

    def consume(ci, b):
        cb = base + ci * CH
        with jax.named_scope("sc_wait"):
            pltpu.make_async_copy(txy_hbm.at[ixy_v.at[b]], rxy_v.at[b],
                                  sems_xy[b]).wait()
            pltpu.make_async_copy(tuv_hbm.at[iuv_v.at[b]], ruv_v.at[b],
                                  sems_uv[b]).wait()

        def ib(g):
            s = g * L
            x, y, u, v, xi, yi, ui, vi = _coords(xy_v, b, s)
            wx = x - xi.astype(jnp.float32)
            wy = y - yi.astype(jnp.float32)
            wu = u - ui.astype(jnp.float32)
            wv = v - vi.astype(jnp.float32)
            w00 = (1.0 - wx) * (1.0 - wy)
            w01 = (1.0 - wx) * wy
            w10 = wx * (1.0 - wy)
            w11 = wx * wy
            a00 = (1.0 - wu) * (1.0 - wv)
            a01 = (1.0 - wu) * wv
            a10 = wu * (1.0 - wv)
            a11 = wu * wv
            rows = s + lax.iota(jnp.int32, L)
            rxy_b = rxy_v.at[b]
            ruv_b = ruv_v.at[b]
            for c in range(C):
                i0 = jnp.full((L,), c, jnp.int32)
                i1 = jnp.full((L,), c + CP, jnp.int32)
                i2 = jnp.full((L,), c + 2 * CP, jnp.int32)
                i3 = jnp.full((L,), c + 3 * CP, jnp.int32)
                fxy = (plsc.load_gather(rxy_b, [rows, i0]) * w00
                       + plsc.load_gather(rxy_b, [rows, i1]) * w01
                       + plsc.load_gather(rxy_b, [rows, i2]) * w10
                       + plsc.load_gather(rxy_b, [rows, i3]) * w11)
                fuv = (plsc.load_gather(ruv_b, [rows, i0]) * a00
                       + plsc.load_gather(ruv_b, [rows, i1]) * a01
                       + plsc.load_gather(ruv_b, [rows, i2]) * a10
                       + plsc.load_gather(ruv_b, [rows, i3]) * a11)
                k0_v[c, pl.ds(s, L)] = fxy * fuv
        with jax.named_scope("sc_interp"):
            plsc.parallel_loop(0, G, unroll=2)(ib)

        with jax.named_scope("sc_k0st"):
            pltpu.sync_copy(k0_v, out_hbm.at[:, pl.ds(cb, CH)])

    stage(0, 0)

    def chunk2(cj, carry):
        for b in range(2):
            ci = cj * 2 + b

            @pl.when(ci + 1 < NCHUNK)
            def _():
                stage(ci + 1, 1 - b)
            consume(ci, b)
        return carry
    lax.fori_loop(0, NCHUNK // 2, chunk2, 0)


@functools.cache
def _sc_interp():
    return functools.partial(
        pl.kernel,
        out_type=jax.ShapeDtypeStruct((CP, N), jnp.float32),
        mesh=plsc.VectorSubcoreMesh(core_axis_name="c", subcore_axis_name="s",
                                    num_cores=NC, num_subcores=NS),
        scratch_types=[
            pltpu.VMEM((2, 4, CH), jnp.float32),
            pltpu.VMEM((2, CH), jnp.int32),
            pltpu.VMEM((2, CH), jnp.int32),
            pltpu.VMEM((2, CH, PACK), jnp.float32),
            pltpu.VMEM((2, CH, PACK), jnp.float32),
            pltpu.VMEM((CP, CH), jnp.float32),
            pltpu.SemaphoreType.DMA,
            pltpu.SemaphoreType.DMA,
            pltpu.SemaphoreType.DMA,
            pltpu.SemaphoreType.DMA,
        ],
        compiler_params=pltpu.CompilerParams(needs_layout_passes=False,
                                             use_tc_tiling_on_sc=False,
                                             disable_bounds_checks=True),
    )(_sc_body)


def _mlp_body(x_ref, w1_ref, b1_ref, w2_ref, b2_ref, w3_ref, b3_ref, o_ref):
    x = x_ref[...]
    h = lax.dot_general(x, w1_ref[...], (((0,), (0,)), ((), ())),
                        preferred_element_type=jnp.float32)
    h = jnp.maximum(h + b1_ref[...], 0.0)
    h = jnp.maximum(jnp.dot(h, w2_ref[...], preferred_element_type=jnp.float32)
                    + b2_ref[...], 0.0)
    o = jnp.dot(h, w3_ref[...], preferred_element_type=jnp.float32) + b3_ref[...]
    o_ref[...] = jax.nn.sigmoid(o)


_mlp = pl.pallas_call(
    _mlp_body,
    grid=(N // BN,),
    in_specs=[
        pl.BlockSpec((CP, BN), lambda i: (0, i)),
        pl.BlockSpec((CP, 128), lambda i: (0, 0)),
        pl.BlockSpec((1, 128), lambda i: (0, 0)),
        pl.BlockSpec((128, 128), lambda i: (0, 0)),
        pl.BlockSpec((1, 128), lambda i: (0, 0)),
        pl.BlockSpec((128, 8), lambda i: (0, 0)),
        pl.BlockSpec((1, 8), lambda i: (0, 0)),
    ],
    out_specs=pl.BlockSpec((BN, 8), lambda i: (i, 0)),
    out_shape=jax.ShapeDtypeStruct((N, 8), jnp.float32),
)


def kernel(xyuv, plane_xy, plane_uv, W1, b1, W2, b2, W3, b3):
    xyuv_t = xyuv.T
    txy = _pack_table(plane_xy)
    tuv = _pack_table(plane_uv)
    k0t = _sc_interp()(xyuv_t, txy, tuv)
    w1p = jnp.pad(W1, ((0, CP - C), (0, 0)))
    w3p = jnp.pad(W3, ((0, 0), (0, 8 - 3)))
    b3p = jnp.pad(b3, (0, 8 - 3))
    out = _mlp(k0t, w1p, b1.reshape(1, -1), W2, b2.reshape(1, -1),
               w3p, b3p.reshape(1, -1))
    return out[:, :3]


# speedup vs baseline: 1.2425x; 1.2170x over previous
import functools

import jax
import jax.numpy as jnp
from jax import lax
from jax.experimental import pallas as pl
from jax.experimental.pallas import tpu as pltpu
from jax.experimental.pallas import tpu_sc as plsc

N = 65536
C = 12
CP = 16
PACK = 4 * CP
NC, NS, L = 2, 16, 16
NW = NC * NS
BPW = N // NW
CH = 128
NCHUNK = BPW // CH
G = CH // L

HXY = 128
HUV = 256
BN = 512


def _pack_table(plane):
    H, W, _ = plane.shape
    p = jnp.pad(plane, ((0, 0), (0, 0), (0, CP - C)))
    t = jnp.concatenate([p[:-1, :-1], p[:-1, 1:], p[1:, :-1], p[1:, 1:]], axis=-1)
    return t.reshape((H - 1) * (W - 1), PACK)


def _sc_body(xyuv_hbm, txy_hbm, tuv_hbm, out_hbm,
             xy_v, ixy_v, iuv_v, wt_v, rxy_v, ruv_v, k0_v,
             sxy0, sxy1, suv0, suv1):
    wid = lax.axis_index("s") * NC + lax.axis_index("c")
    base = wid * BPW
    sems_xy = (sxy0, sxy1)
    sems_uv = (suv0, suv1)

    pltpu.sync_copy(xyuv_hbm.at[:, pl.ds(base, BPW)], xy_v)

    def stage(ci, b):
        o = ci * CH

        def idxb(g):
            s = g * L
            x = jnp.clip(xy_v[0, pl.ds(o + s, L)], 0.0, 1.0) * float(HXY - 1)
            y = jnp.clip(xy_v[1, pl.ds(o + s, L)], 0.0, 1.0) * float(HXY - 1)
            u = jnp.clip(xy_v[2, pl.ds(o + s, L)], 0.0, 1.0) * float(HUV - 1)
            v = jnp.clip(xy_v[3, pl.ds(o + s, L)], 0.0, 1.0) * float(HUV - 1)
            xi = jnp.minimum(x.astype(jnp.int32), HXY - 2)
            yi = jnp.minimum(y.astype(jnp.int32), HXY - 2)
            ui = jnp.minimum(u.astype(jnp.int32), HUV - 2)
            vi = jnp.minimum(v.astype(jnp.int32), HUV - 2)
            ixy_v[b, pl.ds(s, L)] = xi * (HXY - 1) + yi
            iuv_v[b, pl.ds(s, L)] = ui * (HUV - 1) + vi
            wt_v[b, 0, pl.ds(s, L)] = x - xi.astype(jnp.float32)
            wt_v[b, 1, pl.ds(s, L)] = y - yi.astype(jnp.float32)
            wt_v[b, 2, pl.ds(s, L)] = u - ui.astype(jnp.float32)
            wt_v[b, 3, pl.ds(s, L)] = v - vi.astype(jnp.float32)
        with jax.named_scope("sc_idx"):
            plsc.parallel_loop(0, G, unroll=2)(idxb)

        with jax.named_scope("sc_fire"):
            pltpu.async_copy(txy_hbm.at[ixy_v.at[b]], rxy_v.at[b], sems_xy[b])
            pltpu.async_copy(tuv_hbm.at[iuv_v.at[b]], ruv_v.at[b], sems_uv[b])

    def consume(ci, b):
        cb = base + ci * CH
        with jax.named_scope("sc_wait"):
            pltpu.make_async_copy(txy_hbm.at[ixy_v.at[b]], rxy_v.at[b],
                                  sems_xy[b]).wait()
            pltpu.make_async_copy(tuv_hbm.at[iuv_v.at[b]], ruv_v.at[b],
                                  sems_uv[b]).wait()

        def ib(g):
            s = g * L
            wx = wt_v[b, 0, pl.ds(s, L)]
            wy = wt_v[b, 1, pl.ds(s, L)]
            wu = wt_v[b, 2, pl.ds(s, L)]
            wv = wt_v[b, 3, pl.ds(s, L)]
            pxy = ((1.0 - wx) * (1.0 - wy), (1.0 - wx) * wy,
                   wx * (1.0 - wy), wx * wy)
            puv = ((1.0 - wu) * (1.0 - wv), (1.0 - wu) * wv,
                   wu * (1.0 - wv), wu * wv)
            for j in range(L):
                r = s + j
                lane = jnp.full((L,), j, jnp.int32)
                fxy = sum(
                    rxy_v[b, r, pl.ds(k * CP, L)]
                    * jnp.take_along_axis(pxy[k], lane, axis=0,
                                          mode="promise_in_bounds")
                    for k in range(4))
                fuv = sum(
                    ruv_v[b, r, pl.ds(k * CP, L)]
                    * jnp.take_along_axis(puv[k], lane, axis=0,
                                          mode="promise_in_bounds")
                    for k in range(4))
                k0_v[r] = fxy * fuv
        with jax.named_scope("sc_interp"):
            plsc.parallel_loop(0, G, unroll=1)(ib)

        with jax.named_scope("sc_k0st"):
            pltpu.sync_copy(k0_v, out_hbm.at[pl.ds(cb, CH)])

    stage(0, 0)

    def chunk2(cj, carry):
        for b in range(2):
            ci = cj * 2 + b

            @pl.when(ci + 1 < NCHUNK)
            def _():
                stage(ci + 1, 1 - b)
            consume(ci, b)
        return carry
    lax.fori_loop(0, NCHUNK // 2, chunk2, 0)


@functools.cache
def _sc_interp():
    return functools.partial(
        pl.kernel,
        out_type=jax.ShapeDtypeStruct((N, CP), jnp.float32),
        mesh=plsc.VectorSubcoreMesh(core_axis_name="c", subcore_axis_name="s",
                                    num_cores=NC, num_subcores=NS),
        scratch_types=[
            pltpu.VMEM((4, BPW), jnp.float32),
            pltpu.VMEM((2, CH), jnp.int32),
            pltpu.VMEM((2, CH), jnp.int32),
            pltpu.VMEM((2, 4, CH), jnp.float32),
            pltpu.VMEM((2, CH, PACK), jnp.float32),
            pltpu.VMEM((2, CH, PACK), jnp.float32),
            pltpu.VMEM((CH, CP), jnp.float32),
            pltpu.SemaphoreType.DMA,
            pltpu.SemaphoreType.DMA,
            pltpu.SemaphoreType.DMA,
            pltpu.SemaphoreType.DMA,
        ],
        compiler_params=pltpu.CompilerParams(needs_layout_passes=False,
                                             use_tc_tiling_on_sc=False,
                                             disable_bounds_checks=True),
    )(_sc_body)


def _mlp_body(x_ref, w1_ref, b1_ref, w2_ref, b2_ref, w3_ref, b3_ref, o_ref):
    x = x_ref[...]
    h = jnp.maximum(jnp.dot(x, w1_ref[...], preferred_element_type=jnp.float32)
                    + b1_ref[...], 0.0)
    h = jnp.maximum(jnp.dot(h, w2_ref[...], preferred_element_type=jnp.float32)
                    + b2_ref[...], 0.0)
    o = jnp.dot(h, w3_ref[...], preferred_element_type=jnp.float32) + b3_ref[...]
    o_ref[...] = jax.nn.sigmoid(o)


_mlp = pl.pallas_call(
    _mlp_body,
    grid=(N // BN,),
    in_specs=[
        pl.BlockSpec((BN, CP), lambda i: (i, 0)),
        pl.BlockSpec((CP, 128), lambda i: (0, 0)),
        pl.BlockSpec((1, 128), lambda i: (0, 0)),
        pl.BlockSpec((128, 128), lambda i: (0, 0)),
        pl.BlockSpec((1, 128), lambda i: (0, 0)),
        pl.BlockSpec((128, 8), lambda i: (0, 0)),
        pl.BlockSpec((1, 8), lambda i: (0, 0)),
    ],
    out_specs=pl.BlockSpec((BN, 8), lambda i: (i, 0)),
    out_shape=jax.ShapeDtypeStruct((N, 8), jnp.float32),
)


def kernel(xyuv, plane_xy, plane_uv, W1, b1, W2, b2, W3, b3):
    xyuv_t = xyuv.T
    txy = _pack_table(plane_xy)
    tuv = _pack_table(plane_uv)
    k0 = _sc_interp()(xyuv_t, txy, tuv)
    w1p = jnp.pad(W1, ((0, CP - C), (0, 0)))
    w3p = jnp.pad(W3, ((0, 0), (0, 8 - 3)))
    b3p = jnp.pad(b3, (0, 8 - 3))
    out = _mlp(k0, w1p, b1.reshape(1, -1), W2, b2.reshape(1, -1),
               w3p, b3p.reshape(1, -1))
    return out[:, :3]
